# Initial kernel scaffold; baseline (speedup 1.0000x reference)
import jax
import jax.numpy as jnp
from jax.experimental import pallas as pl


def kernel(x, edge_index, pos_edge_index, neg_edge_index, W1, b1, W2, b2, Wl, bl):
    raise NotImplementedError("write your pallas kernel here")


# speedup vs baseline: 9.8633x; 9.8633x over previous
import jax
import jax.numpy as jnp
from jax import lax
from jax.experimental import pallas as pl
from jax.experimental.pallas import tpu as pltpu
from jax.experimental.pallas import tpu_sc as plsc

N = 10000
E = 320000
D = 128
H = 64

NC = 2
NS = 16
NW = NC * NS
EW = E // NW
K = 80
CH = EW // K
ET = 2 * E
EDW = ET // NW
CHD = EDW // K
NPA = 624
NREM = N - NS * NPA

_MESH = plsc.VectorSubcoreMesh(
    core_axis_name="c", subcore_axis_name="s", num_cores=NC, num_subcores=NS
)
_SC_PARAMS = pltpu.CompilerParams(
    needs_layout_passes=False, use_tc_tiling_on_sc=False
)

f32 = jnp.float32
i32 = jnp.int32


def _deg_body(dst_hbm, out_hbm, idx_v, acc_v, sem):
    c = lax.axis_index("c")
    s = lax.axis_index("s")
    w = c * NS + s
    cp = pltpu.async_copy(dst_hbm.at[pl.ds(w * EW, EW)], idx_v, sem)

    def zero(i, _):
        acc_v[pl.ds(i * 16, 16)] = jnp.zeros((16,), f32)
        return 0

    lax.fori_loop(0, N // 16, zero, 0)
    cp.wait()
    ones = jnp.ones((16,), f32)

    def body(i, _):
        idx = idx_v[pl.ds(i * 16, 16)]
        plsc.addupdate_scatter(acc_v, [idx], ones)
        return 0

    lax.fori_loop(0, EW // 16, body, 0)
    pltpu.sync_copy(acc_v, out_hbm.at[pl.ds(w * N, N)])


_deg = pl.kernel(
    _deg_body,
    out_type=jax.ShapeDtypeStruct((NW * N,), f32),
    mesh=_MESH,
    scratch_types=[
        pltpu.VMEM((EW,), i32),
        pltpu.VMEM((N,), f32),
        pltpu.SemaphoreType.DMA,
    ],
    compiler_params=_SC_PARAMS,
)


def _agg_body(srce, dste, table, zer, out_hbm, idx_s, idx_d, rows, acc_sh, sem):
    c = lax.axis_index("c")
    s = lax.axis_index("s")
    w = c * NS + s

    @pl.when(s == 0)
    def _():
        pltpu.sync_copy(zer, acc_sh)

    plsc.subcore_barrier()

    def chunk(g, _):
        base = w * EW + g * K
        pltpu.sync_copy(srce.at[pl.ds(base, K)], idx_s)
        pltpu.sync_copy(dste.at[pl.ds(base, K)], idx_d)
        pltpu.async_copy(table.at[idx_s], rows, sem).wait()
        pltpu.sync_copy(rows, acc_sh.at[idx_d], add=True)
        return 0

    lax.fori_loop(0, CH, chunk, 0)
    plsc.subcore_barrier()
    pltpu.sync_copy(
        acc_sh.at[pl.ds(s * NPA, NPA)], out_hbm.at[c, pl.ds(s * NPA, NPA)]
    )

    @pl.when(s == 0)
    def _():
        pltpu.sync_copy(
            acc_sh.at[pl.ds(NS * NPA, NREM)], out_hbm.at[c, pl.ds(NS * NPA, NREM)]
        )


_agg = pl.kernel(
    _agg_body,
    out_type=jax.ShapeDtypeStruct((NC, N, H), f32),
    mesh=_MESH,
    scratch_types=[
        pltpu.VMEM((K,), i32),
        pltpu.VMEM((K,), i32),
        pltpu.VMEM((K, H), f32),
        pltpu.VMEM_SHARED((N, H), f32),
        pltpu.SemaphoreType.DMA,
    ],
    compiler_params=_SC_PARAMS,
)


def _dec_body(srce, dste, a_hbm, z_hbm, bl_hbm, out_hbm,
              idx_s, idx_d, rows_a, rows_z, sc_buf, tmp_v, bl_v, sem, sem2):
    c = lax.axis_index("c")
    s = lax.axis_index("s")
    w = c * NS + s
    pltpu.sync_copy(bl_hbm, bl_v)
    blvec = bl_v[...]
    lanes16 = lax.iota(i32, 16) * 16

    def chunk(g, _):
        base = w * EDW + g * K
        pltpu.sync_copy(srce.at[pl.ds(base, K)], idx_s)
        pltpu.sync_copy(dste.at[pl.ds(base, K)], idx_d)
        cp1 = pltpu.async_copy(a_hbm.at[idx_s], rows_a, sem)
        cp2 = pltpu.async_copy(z_hbm.at[idx_d], rows_z, sem2)
        cp1.wait()
        cp2.wait()

        def group(j, _):
            def edge(e, _):
                eg = j * 16 + e
                acc = rows_a[eg, pl.ds(0, 16)] * rows_z[eg, pl.ds(0, 16)]
                acc = acc + rows_a[eg, pl.ds(16, 16)] * rows_z[eg, pl.ds(16, 16)]
                acc = acc + rows_a[eg, pl.ds(32, 16)] * rows_z[eg, pl.ds(32, 16)]
                acc = acc + rows_a[eg, pl.ds(48, 16)] * rows_z[eg, pl.ds(48, 16)]
                plsc.store_scatter(tmp_v, [lanes16 + e], acc)
                return 0

            lax.fori_loop(0, 16, edge, 0)

            def srow(l, sv):
                return sv + tmp_v[pl.ds(l * 16, 16)]

            sv = lax.fori_loop(0, 16, srow, blvec)
            sc_buf[pl.ds(j * 16, 16)] = sv
            return 0

        lax.fori_loop(0, K // 16, group, 0)
        pltpu.sync_copy(sc_buf, out_hbm.at[pl.ds(base, K)])
        return 0

    lax.fori_loop(0, CHD, chunk, 0)


_dec = pl.kernel(
    _dec_body,
    out_type=jax.ShapeDtypeStruct((ET,), f32),
    mesh=_MESH,
    scratch_types=[
        pltpu.VMEM((K,), i32),
        pltpu.VMEM((K,), i32),
        pltpu.VMEM((K, H), f32),
        pltpu.VMEM((K, H), f32),
        pltpu.VMEM((K,), f32),
        pltpu.VMEM((256,), f32),
        pltpu.VMEM((16,), f32),
        pltpu.SemaphoreType.DMA,
        pltpu.SemaphoreType.DMA,
    ],
    compiler_params=_SC_PARAMS,
)


def _tc1_body(x_ref, w1_ref, degp_ref, xs_ref, dis_ref):
    deg2 = jnp.sum(degp_ref[...], axis=0, keepdims=True) + 1.0
    dis2 = lax.rsqrt(deg2)
    dis_ref[...] = dis2
    discol = dis2.T
    xw = jnp.dot(x_ref[...], w1_ref[...], preferred_element_type=f32)
    xs_ref[...] = xw * discol


_tc1 = pl.pallas_call(
    _tc1_body,
    out_shape=[
        jax.ShapeDtypeStruct((N, H), f32),
        jax.ShapeDtypeStruct((1, N), f32),
    ],
)


def _tc2_body(aggp_ref, xs_ref, dis_ref, b1_ref, w2_ref, hs_ref):
    ssum = aggp_ref[0] + aggp_ref[1] + xs_ref[...]
    discol = dis_ref[...].T
    h = jnp.maximum(ssum * discol + b1_ref[...], 0.0)
    hs_ref[...] = jnp.dot(h, w2_ref[...], preferred_element_type=f32) * discol


_tc2 = pl.pallas_call(
    _tc2_body,
    out_shape=[jax.ShapeDtypeStruct((N, H), f32)],
)


def _tc3_body(aggp_ref, hs_ref, dis_ref, b2_ref, wl_ref, z_ref, a_ref):
    ssum = aggp_ref[0] + aggp_ref[1] + hs_ref[...]
    discol = dis_ref[...].T
    z = jnp.maximum(ssum * discol + b2_ref[...], 0.0)
    z_ref[...] = z
    a_ref[...] = z * wl_ref[...]


_tc3 = pl.pallas_call(
    _tc3_body,
    out_shape=[
        jax.ShapeDtypeStruct((N, H), f32),
        jax.ShapeDtypeStruct((N, H), f32),
    ],
)


def kernel(x, edge_index, pos_edge_index, neg_edge_index, W1, b1, W2, b2, Wl, bl):
    src = edge_index[0].reshape(-1)
    dst = edge_index[1].reshape(-1)
    zer = jnp.zeros((N, H), f32)

    degp = _deg(dst).reshape(NW, N)
    xs, dis2 = _tc1(x, W1, degp)
    aggp1 = _agg(src, dst, xs, zer)
    (hs,) = _tc2(aggp1, xs, dis2, b1.reshape(1, H), W2)
    aggp2 = _agg(src, dst, hs, zer)
    z, a = _tc3(aggp2, hs, dis2, b2.reshape(1, H), Wl.reshape(1, H))

    srcd = jnp.concatenate([pos_edge_index[0], neg_edge_index[0]])
    dstd = jnp.concatenate([pos_edge_index[1], neg_edge_index[1]])
    bl16 = jnp.broadcast_to(bl, (16,)).astype(f32)
    scores = _dec(srcd, dstd, a, z, bl16)
    return scores[:E], scores[E:]
